# Initial kernel scaffold; baseline (speedup 1.0000x reference)
import jax
import jax.numpy as jnp
from jax.experimental import pallas as pl


def kernel(sparse_feats, labels, global_step, W1, b1, W2, b2, prototypes):
    raise NotImplementedError("write your pallas kernel here")


# speedup vs baseline: 4.9388x; 4.9388x over previous
import functools

import jax
import jax.numpy as jnp
from jax.experimental import pallas as pl
from jax.experimental.pallas import tpu as pltpu

N = 100000
D_IN = 128
P = 256
C = 20
MOMENTUM = 0.99

R = 2000
GRID = N // R


def _body(m_ref, labels_ref, x_ref, w1_ref, b1_ref, w2_ref, b2_ref, proto_ref,
          scores_ref, feats_ref, newproto_ref, sums_scr, cnts_scr):
    step = pl.program_id(0)

    feats = jnp.dot(x_ref[...], w1_ref[...],
                    preferred_element_type=jnp.float32) + b1_ref[...]
    feats_ref[...] = feats
    scores_ref[...] = jnp.dot(feats, w2_ref[...],
                              preferred_element_type=jnp.float32) + b2_ref[...]

    lab = labels_ref[0, 0, :]
    cls = jax.lax.broadcasted_iota(jnp.int32, (C, R), 0)
    onehot_t = (lab[None, :] == cls).astype(jnp.float32)
    psum = jax.lax.dot(onehot_t, feats,
                       precision=jax.lax.Precision.HIGHEST,
                       preferred_element_type=jnp.float32)
    pcnt = jnp.sum(onehot_t, axis=1, keepdims=True)

    @pl.when(step == 0)
    def _init():
        sums_scr[...] = psum
        cnts_scr[...] = jnp.broadcast_to(pcnt, (C, 128))

    @pl.when(step != 0)
    def _acc():
        sums_scr[...] += psum
        cnts_scr[...] += jnp.broadcast_to(pcnt, (C, 128))

    @pl.when(step == GRID - 1)
    def _finish():
        m = m_ref[0, 0]
        cnt = cnts_scr[:, :1]
        means = sums_scr[...] / jnp.maximum(cnt, 1.0)
        proto = proto_ref[...]
        newproto_ref[...] = jnp.where(cnt > 0.0,
                                      proto * m + means * (1.0 - m), proto)


@functools.partial(jax.jit, static_argnames=())
def kernel(sparse_feats, labels, global_step, W1, b1, W2, b2, prototypes):
    gs = jnp.asarray(global_step).astype(jnp.float32)
    m = jnp.minimum(1.0 - 1.0 / (1.0 + gs), MOMENTUM).reshape(1, 1)
    labels3 = labels.reshape(GRID, 1, R)

    scores, feats, newproto = pl.pallas_call(
        _body,
        grid=(GRID,),
        in_specs=[
            pl.BlockSpec(memory_space=pltpu.SMEM),
            pl.BlockSpec((1, 1, R), lambda i: (i, 0, 0)),
            pl.BlockSpec((R, D_IN), lambda i: (i, 0)),
            pl.BlockSpec((D_IN, P), lambda i: (0, 0)),
            pl.BlockSpec((1, P), lambda i: (0, 0)),
            pl.BlockSpec((P, C), lambda i: (0, 0)),
            pl.BlockSpec((1, C), lambda i: (0, 0)),
            pl.BlockSpec((C, P), lambda i: (0, 0)),
        ],
        out_specs=[
            pl.BlockSpec((R, C), lambda i: (i, 0)),
            pl.BlockSpec((R, P), lambda i: (i, 0)),
            pl.BlockSpec((C, P), lambda i: (0, 0)),
        ],
        out_shape=[
            jax.ShapeDtypeStruct((N, C), jnp.float32),
            jax.ShapeDtypeStruct((N, P), jnp.float32),
            jax.ShapeDtypeStruct((C, P), jnp.float32),
        ],
        scratch_shapes=[
            pltpu.VMEM((C, P), jnp.float32),
            pltpu.VMEM((C, 128), jnp.float32),
        ],
    )(m, labels3, sparse_feats, W1, b1.reshape(1, P), W2, b2.reshape(1, C),
      prototypes)

    return scores, feats, prototypes, newproto


# speedup vs baseline: 6.7306x; 1.3628x over previous
import functools

import jax
import jax.numpy as jnp
from jax.experimental import pallas as pl
from jax.experimental.pallas import tpu as pltpu

N = 100000
D_IN = 128
P = 256
C = 20
MOMENTUM = 0.99

R = 5000
GRID = N // R


def _body(m_ref, labels_ref, x_ref, w1_ref, b1_ref, w2_ref, b2_ref, proto_ref,
          scores_ref, feats_ref, newproto_ref, sums_scr, cnts_scr):
    step = pl.program_id(0)

    x = x_ref[...]
    feats = jnp.dot(x, w1_ref[...],
                    preferred_element_type=jnp.float32) + b1_ref[...]
    feats_ref[...] = feats
    scores_ref[...] = jnp.dot(feats, w2_ref[...],
                              preferred_element_type=jnp.float32) + b2_ref[...]

    lab = labels_ref[0, 0, :]
    cls = jax.lax.broadcasted_iota(jnp.int32, (C, R), 0)
    onehot_t = (lab[None, :] == cls).astype(jnp.float32)
    psum = jnp.dot(onehot_t, x, preferred_element_type=jnp.float32)
    pcnt = jnp.sum(onehot_t, axis=1, keepdims=True)

    @pl.when(step == 0)
    def _init():
        sums_scr[...] = psum
        cnts_scr[...] = jnp.broadcast_to(pcnt, (C, 128))

    @pl.when(step != 0)
    def _acc():
        sums_scr[...] += psum
        cnts_scr[...] += jnp.broadcast_to(pcnt, (C, 128))

    @pl.when(step == GRID - 1)
    def _finish():
        m = m_ref[0, 0]
        cnt = cnts_scr[:, :1]
        means_x = sums_scr[...] / jnp.maximum(cnt, 1.0)
        means = jnp.dot(means_x, w1_ref[...],
                        preferred_element_type=jnp.float32) + b1_ref[...]
        proto = proto_ref[...]
        newproto_ref[...] = jnp.where(cnt > 0.0,
                                      proto * m + means * (1.0 - m), proto)


@functools.partial(jax.jit, static_argnames=())
def kernel(sparse_feats, labels, global_step, W1, b1, W2, b2, prototypes):
    gs = jnp.asarray(global_step).astype(jnp.float32)
    m = jnp.minimum(1.0 - 1.0 / (1.0 + gs), MOMENTUM).reshape(1, 1)
    labels3 = labels.reshape(GRID, 1, R)

    scores, feats, newproto = pl.pallas_call(
        _body,
        grid=(GRID,),
        in_specs=[
            pl.BlockSpec(memory_space=pltpu.SMEM),
            pl.BlockSpec((1, 1, R), lambda i: (i, 0, 0)),
            pl.BlockSpec((R, D_IN), lambda i: (i, 0)),
            pl.BlockSpec((D_IN, P), lambda i: (0, 0)),
            pl.BlockSpec((1, P), lambda i: (0, 0)),
            pl.BlockSpec((P, C), lambda i: (0, 0)),
            pl.BlockSpec((1, C), lambda i: (0, 0)),
            pl.BlockSpec((C, P), lambda i: (0, 0)),
        ],
        out_specs=[
            pl.BlockSpec((R, C), lambda i: (i, 0)),
            pl.BlockSpec((R, P), lambda i: (i, 0)),
            pl.BlockSpec((C, P), lambda i: (0, 0)),
        ],
        out_shape=[
            jax.ShapeDtypeStruct((N, C), jnp.float32),
            jax.ShapeDtypeStruct((N, P), jnp.float32),
            jax.ShapeDtypeStruct((C, P), jnp.float32),
        ],
        scratch_shapes=[
            pltpu.VMEM((C, D_IN), jnp.float32),
            pltpu.VMEM((C, 128), jnp.float32),
        ],
    )(m, labels3, sparse_feats, W1, b1.reshape(1, P), W2, b2.reshape(1, C),
      prototypes)

    return scores, feats, prototypes, newproto


# speedup vs baseline: 6.9051x; 1.0259x over previous
import functools

import jax
import jax.numpy as jnp
from jax.experimental import pallas as pl
from jax.experimental.pallas import tpu as pltpu

N = 100000
D_IN = 128
P = 256
C = 20
MOMENTUM = 0.99

R = 10000
GRID = N // R


def _body(m_ref, labels_ref, x_ref, w1_ref, b1_ref, w2_ref, b2_ref, proto_ref,
          scores_ref, feats_ref, newproto_ref, sums_scr, cnts_scr):
    step = pl.program_id(0)

    x = x_ref[...]
    feats = jnp.dot(x, w1_ref[...],
                    preferred_element_type=jnp.float32) + b1_ref[...]
    feats_ref[...] = feats
    scores_ref[...] = jnp.dot(feats, w2_ref[...],
                              preferred_element_type=jnp.float32) + b2_ref[...]

    lab = labels_ref[0, 0, :]
    cls = jax.lax.broadcasted_iota(jnp.int32, (C, R), 0)
    onehot_t = (lab[None, :] == cls).astype(jnp.float32)
    psum = jnp.dot(onehot_t, x, preferred_element_type=jnp.float32)
    pcnt = jnp.sum(onehot_t, axis=1, keepdims=True)

    @pl.when(step == 0)
    def _init():
        sums_scr[...] = psum
        cnts_scr[...] = jnp.broadcast_to(pcnt, (C, 128))

    @pl.when(step != 0)
    def _acc():
        sums_scr[...] += psum
        cnts_scr[...] += jnp.broadcast_to(pcnt, (C, 128))

    @pl.when(step == GRID - 1)
    def _finish():
        m = m_ref[0, 0]
        cnt = cnts_scr[:, :1]
        means_x = sums_scr[...] / jnp.maximum(cnt, 1.0)
        means = jnp.dot(means_x, w1_ref[...],
                        preferred_element_type=jnp.float32) + b1_ref[...]
        proto = proto_ref[...]
        newproto_ref[...] = jnp.where(cnt > 0.0,
                                      proto * m + means * (1.0 - m), proto)


@functools.partial(jax.jit, static_argnames=())
def kernel(sparse_feats, labels, global_step, W1, b1, W2, b2, prototypes):
    gs = jnp.asarray(global_step).astype(jnp.float32)
    m = jnp.minimum(1.0 - 1.0 / (1.0 + gs), MOMENTUM).reshape(1, 1)
    labels3 = labels.reshape(GRID, 1, R)

    scores, feats, newproto = pl.pallas_call(
        _body,
        grid=(GRID,),
        in_specs=[
            pl.BlockSpec(memory_space=pltpu.SMEM),
            pl.BlockSpec((1, 1, R), lambda i: (i, 0, 0)),
            pl.BlockSpec((R, D_IN), lambda i: (i, 0)),
            pl.BlockSpec((D_IN, P), lambda i: (0, 0)),
            pl.BlockSpec((1, P), lambda i: (0, 0)),
            pl.BlockSpec((P, C), lambda i: (0, 0)),
            pl.BlockSpec((1, C), lambda i: (0, 0)),
            pl.BlockSpec((C, P), lambda i: (0, 0)),
        ],
        out_specs=[
            pl.BlockSpec((R, C), lambda i: (i, 0)),
            pl.BlockSpec((R, P), lambda i: (i, 0)),
            pl.BlockSpec((C, P), lambda i: (0, 0)),
        ],
        out_shape=[
            jax.ShapeDtypeStruct((N, C), jnp.float32),
            jax.ShapeDtypeStruct((N, P), jnp.float32),
            jax.ShapeDtypeStruct((C, P), jnp.float32),
        ],
        scratch_shapes=[
            pltpu.VMEM((C, D_IN), jnp.float32),
            pltpu.VMEM((C, 128), jnp.float32),
        ],
    )(m, labels3, sparse_feats, W1, b1.reshape(1, P), W2, b2.reshape(1, C),
      prototypes)

    return scores, feats, prototypes, newproto
